# baseline (speedup 1.0000x reference)
import jax
import jax.numpy as jnp
from jax import lax
from jax.experimental import pallas as pl
from jax.experimental.pallas import tpu as pltpu
from jax.experimental.pallas import tpu_sc as plsc

N = 10000
D = 128
E = 320000
NC, NS, L = 2, 16, 16
NW = NC * NS
K = 32
NCHUNK = 320
EPW = NCHUNK * K
EPAD = NW * EPW
NPAD = 10240
RPW = NPAD // NS
W = D + L
IROWS = NCHUNK * L // 128


def _prep_body(beta_ref, x_ref, tabd_ref, bvec_ref):
    x = x_ref[...]
    s2 = jnp.sum(x * x, axis=1, keepdims=True)
    nrm = jnp.sqrt(s2)
    xn = x / jnp.maximum(nrm, 1e-12)
    tabd_ref[0:N, :] = xn
    tabd_ref[N:NPAD, :] = jnp.zeros((NPAD - N, D), jnp.float32)
    b = beta_ref[0]
    bvec_ref[...] = jnp.zeros((8, 128), jnp.float32)
    bvec_ref[0:1, :] = jnp.full((1, 128), b, jnp.float32)
    bvec_ref[1:2, :] = jnp.full((1, 128), -jnp.abs(b), jnp.float32)


_prep = pl.pallas_call(
    _prep_body,
    out_shape=(
        jax.ShapeDtypeStruct((NPAD, D), jnp.float32),
        jax.ShapeDtypeStruct((8, 128), jnp.float32),
    ),
    in_specs=[
        pl.BlockSpec(memory_space=pltpu.SMEM),
        pl.BlockSpec(memory_space=pltpu.VMEM),
    ],
    out_specs=(
        pl.BlockSpec(memory_space=pltpu.VMEM),
        pl.BlockSpec(memory_space=pltpu.VMEM),
    ),
)


def _edge_body(x_hbm, tabd_hbm, bvec_hbm, src_hbm, dst_hbm, out_hbm,
               sidx_all, didx_all,
               sbuf0, sbuf1, dbuf0, dbuf1, wbuf0, wbuf1,
               sidx_u0, sidx_u1, didx_g0, didx_g1, didx_s0, didx_s1,
               bvecv, accum,
               sgs0, sgd0, ssc0, sgs1, sgd1, ssc1):
    c = lax.axis_index("c")
    s = lax.axis_index("s")
    wid = c * NS + s

    pltpu.sync_copy(bvec_hbm, bvecv)
    bvec = bvecv[0, pl.ds(0, L)]
    nbvec = bvecv[1, pl.ds(0, L)]

    pltpu.sync_copy(src_hbm.at[wid], sidx_all)
    pltpu.sync_copy(dst_hbm.at[wid], didx_all)

    zero = jnp.zeros((L,), jnp.float32)

    def zrow(j, carry):
        for t in range(W // L):
            wbuf0[j, pl.ds(t * L, L)] = zero
        return carry

    lax.fori_loop(0, K, zrow, 0)
    for b in range(RPW // K):
        pltpu.sync_copy(wbuf0, accum.at[pl.ds(s * RPW + b * K, K), :])
    plsc.subcore_barrier()

    lomask = jnp.full((L,), 0xFFFF, jnp.int32)
    sh16 = jnp.full((L,), 16, jnp.int32)
    ones16 = jnp.ones((L,), jnp.float32)
    magic = jnp.full((L,), 0x5F3759DF, jnp.int32)
    sh1 = jnp.full((L,), 1, jnp.int32)
    half = jnp.full((L,), 0.5, jnp.float32)
    threehalf = jnp.full((L,), 1.5, jnp.float32)

    def expand(ci, packed_all, ubuf):
        v = packed_all[ci // 8, pl.ds((ci % 8) * L, L)]
        ubuf[0, pl.ds(0, L)] = v & lomask
        ubuf[0, pl.ds(L, L)] = lax.shift_right_logical(v, sh16)

    def gstart(ci, sidx_u, didx_g, sbuf, dbuf, sem_s, sem_d):
        expand(ci, sidx_all, sidx_u)
        expand(ci, didx_all, didx_g)
        pltpu.async_copy(x_hbm.at[sidx_u.at[0]], sbuf, sem_s)
        pltpu.async_copy(tabd_hbm.at[didx_g.at[0]], dbuf, sem_d)

    def gwait(sbuf, dbuf, sem_s, sem_d):
        pltpu.make_async_copy(x_hbm.at[pl.ds(0, K), :], sbuf, sem_s).wait()
        pltpu.make_async_copy(tabd_hbm.at[pl.ds(0, K), :], dbuf, sem_d).wait()

    def sstart(ci, wbuf, didx_s, sem):
        expand(ci, didx_all, didx_s)
        pltpu.async_copy(wbuf, accum.at[didx_s.at[0]], sem, add=True)

    def swait(wbuf, sem):
        pltpu.make_async_copy(out_hbm.at[0, pl.ds(0, K), :], wbuf, sem).wait()

    def compute(sbuf, dbuf, wbuf):
        @plsc.parallel_loop(0, K, unroll=4)
        def edge(e):
            sv = [sbuf[e, pl.ds(t * L, L)] for t in range(D // L)]
            dv = [dbuf[e, pl.ds(t * L, L)] for t in range(D // L)]
            acc0 = sv[0] * dv[0]
            acc1 = sv[1] * dv[1]
            acc2 = sv[2] * dv[2]
            acc3 = sv[3] * dv[3]
            ss0 = sv[0] * sv[0]
            ss1 = sv[1] * sv[1]
            ss2 = sv[2] * sv[2]
            ss3 = sv[3] * sv[3]
            for t in range(4, D // L, 4):
                acc0 = acc0 + sv[t] * dv[t]
                acc1 = acc1 + sv[t + 1] * dv[t + 1]
                acc2 = acc2 + sv[t + 2] * dv[t + 2]
                acc3 = acc3 + sv[t + 3] * dv[t + 3]
                ss0 = ss0 + sv[t] * sv[t]
                ss1 = ss1 + sv[t + 1] * sv[t + 1]
                ss2 = ss2 + sv[t + 2] * sv[t + 2]
                ss3 = ss3 + sv[t + 3] * sv[t + 3]
            dot = jnp.sum((acc0 + acc1) + (acc2 + acc3))
            ss = jnp.sum((ss0 + ss1) + (ss2 + ss3))
            ssv = jnp.maximum(ss * ones16, 1e-24)
            y = lax.bitcast_convert_type(
                magic - lax.shift_right_logical(
                    lax.bitcast_convert_type(ssv, jnp.int32), sh1),
                jnp.float32)
            hs = half * ssv
            y = y * (threehalf - hs * y * y)
            y = y * (threehalf - hs * y * y)
            y = y * (threehalf - hs * y * y)
            w = jnp.exp((dot * bvec) * y + nbvec)
            for t in range(D // L):
                wbuf[e, pl.ds(t * L, L)] = sv[t] * w
            wbuf[e, pl.ds(D, L)] = w

    gstart(0, sidx_u0, didx_g0, sbuf0, dbuf0, sgs0, sgd0)

    gwait(sbuf0, dbuf0, sgs0, sgd0)
    gstart(1, sidx_u1, didx_g1, sbuf1, dbuf1, sgs1, sgd1)
    compute(sbuf0, dbuf0, wbuf0)
    sstart(0, wbuf0, didx_s0, ssc0)

    gwait(sbuf1, dbuf1, sgs1, sgd1)
    gstart(2, sidx_u0, didx_g0, sbuf0, dbuf0, sgs0, sgd0)
    compute(sbuf1, dbuf1, wbuf1)
    sstart(1, wbuf1, didx_s1, ssc1)

    def pair(i, carry):
        ci = 2 * i
        gwait(sbuf0, dbuf0, sgs0, sgd0)
        swait(wbuf0, ssc0)
        gstart(ci + 1, sidx_u1, didx_g1, sbuf1, dbuf1, sgs1, sgd1)
        compute(sbuf0, dbuf0, wbuf0)
        sstart(ci, wbuf0, didx_s0, ssc0)

        gwait(sbuf1, dbuf1, sgs1, sgd1)
        swait(wbuf1, ssc1)
        gstart(ci + 2, sidx_u0, didx_g0, sbuf0, dbuf0, sgs0, sgd0)
        compute(sbuf1, dbuf1, wbuf1)
        sstart(ci + 1, wbuf1, didx_s1, ssc1)
        return carry

    lax.fori_loop(1, NCHUNK // 2 - 1, pair, 0)

    gwait(sbuf0, dbuf0, sgs0, sgd0)
    swait(wbuf0, ssc0)
    gstart(NCHUNK - 1, sidx_u1, didx_g1, sbuf1, dbuf1, sgs1, sgd1)
    compute(sbuf0, dbuf0, wbuf0)
    sstart(NCHUNK - 2, wbuf0, didx_s0, ssc0)

    gwait(sbuf1, dbuf1, sgs1, sgd1)
    swait(wbuf1, ssc1)
    compute(sbuf1, dbuf1, wbuf1)
    sstart(NCHUNK - 1, wbuf1, didx_s1, ssc1)

    swait(wbuf0, ssc0)
    swait(wbuf1, ssc1)
    plsc.subcore_barrier()
    pltpu.sync_copy(accum.at[pl.ds(s * RPW, RPW), :],
                    out_hbm.at[c, pl.ds(s * RPW, RPW), :])


_edge = pl.kernel(
    _edge_body,
    out_type=jax.ShapeDtypeStruct((NC, NPAD, W), jnp.float32),
    mesh=plsc.VectorSubcoreMesh(core_axis_name="c", subcore_axis_name="s"),
    compiler_params=pltpu.CompilerParams(
        needs_layout_passes=False, use_tc_tiling_on_sc=False),
    scratch_types=[
        pltpu.VMEM((IROWS, 128), jnp.int32),
        pltpu.VMEM((IROWS, 128), jnp.int32),
        pltpu.VMEM((K, D), jnp.float32),
        pltpu.VMEM((K, D), jnp.float32),
        pltpu.VMEM((K, D), jnp.float32),
        pltpu.VMEM((K, D), jnp.float32),
        pltpu.VMEM((K, W), jnp.float32),
        pltpu.VMEM((K, W), jnp.float32),
        pltpu.VMEM((1, K), jnp.int32),
        pltpu.VMEM((1, K), jnp.int32),
        pltpu.VMEM((1, K), jnp.int32),
        pltpu.VMEM((1, K), jnp.int32),
        pltpu.VMEM((1, K), jnp.int32),
        pltpu.VMEM((1, K), jnp.int32),
        pltpu.VMEM((8, 128), jnp.float32),
        pltpu.VMEM_SHARED((NPAD, W), jnp.float32),
        pltpu.SemaphoreType.DMA,
        pltpu.SemaphoreType.DMA,
        pltpu.SemaphoreType.DMA,
        pltpu.SemaphoreType.DMA,
        pltpu.SemaphoreType.DMA,
        pltpu.SemaphoreType.DMA,
    ],
)


def _combine_body(beta_ref, x_ref, p_ref, o_ref):
    x = x_ref[...]
    b = beta_ref[0]
    s2 = jnp.sum(x * x, axis=1, keepdims=True)
    nrm = jnp.maximum(jnp.sqrt(s2), 1e-12)
    xn2 = s2 / (nrm * nrm)
    selfw = jnp.exp(b * xn2 - jnp.abs(b))
    num = p_ref[0, 0:N, 0:D] + p_ref[1, 0:N, 0:D] + selfw * x
    den = p_ref[0, 0:N, D:D + 1] + p_ref[1, 0:N, D:D + 1] + selfw
    o_ref[...] = num / den


_combine = pl.pallas_call(
    _combine_body,
    out_shape=jax.ShapeDtypeStruct((N, D), jnp.float32),
    in_specs=[
        pl.BlockSpec(memory_space=pltpu.SMEM),
        pl.BlockSpec(memory_space=pltpu.VMEM),
        pl.BlockSpec(memory_space=pltpu.VMEM),
    ],
    out_specs=pl.BlockSpec(memory_space=pltpu.VMEM),
)


def _pack_idx(ids):
    h = ids.astype(jnp.int16).reshape(NW, NCHUNK, 2, L).swapaxes(-2, -1)
    return lax.bitcast_convert_type(h, jnp.int32).reshape(NW, IROWS, 128)


def kernel(x, edge_index, beta):
    tabd, bvec = _prep(beta, x)
    pads = jnp.zeros((EPAD - E,), jnp.int32)
    padd = jnp.full((EPAD - E,), NPAD - 1, jnp.int32)
    srcp = _pack_idx(jnp.concatenate([edge_index[0], pads]))
    dstp = _pack_idx(jnp.concatenate([edge_index[1], padd]))
    partials = _edge(x, tabd, bvec, srcp, dstp)
    return _combine(beta, x, partials)


# speedup vs baseline: 2.3054x; 2.3054x over previous
import jax
import jax.numpy as jnp
from jax import lax
from jax.experimental import pallas as pl
from jax.experimental.pallas import tpu as pltpu
from jax.experimental.pallas import tpu_sc as plsc

N = 10000
D = 128
E = 320000
NC, NS, L = 2, 16, 16
NW = NC * NS
K = 32
NCHUNK = 320
EPW = NCHUNK * K
EPAD = NW * EPW
NPAD = 10240
RPW = NPAD // NS
W = D + L
IROWS = NCHUNK * L // 128
WROW = EPAD // 256
WREAL = E // 256


def _prep_body(beta_ref, x_ref, ei_ref, tabd_ref, bvec_ref, srcp_ref, dstp_ref):
    x = x_ref[...]
    s2 = jnp.sum(x * x, axis=1, keepdims=True)
    nrm = jnp.sqrt(s2)
    xn = x / jnp.maximum(nrm, 1e-12)
    tabd_ref[0:N, :] = xn
    tabd_ref[N:NPAD, :] = jnp.zeros((NPAD - N, D), jnp.float32)
    b = beta_ref[0]
    bvec_ref[...] = jnp.zeros((8, 128), jnp.float32)
    bvec_ref[0:1, :] = jnp.full((1, 128), b, jnp.float32)
    bvec_ref[1:2, :] = jnp.full((1, 128), -jnp.abs(b), jnp.float32)
    for half, ref in ((0, srcp_ref), (1, dstp_ref)):
        v3 = ei_ref[half].reshape(WREAL, 2, 128)
        ref[0:WREAL, :] = v3[:, 0, :] | (v3[:, 1, :] << 16)
    f = (lax.broadcasted_iota(jnp.int32, (WROW - WREAL, 128), 0) * 128
         + lax.broadcasted_iota(jnp.int32, (WROW - WREAL, 128), 1))
    srcp_ref[WREAL:WROW, :] = ((2 * f) % 9973) | (((2 * f + 1) % 9973) << 16)
    dstp_ref[WREAL:WROW, :] = ((N + (2 * f) % (NPAD - N))
                               | ((N + (2 * f + 1) % (NPAD - N)) << 16))


_prep = pl.pallas_call(
    _prep_body,
    out_shape=(
        jax.ShapeDtypeStruct((NPAD, D), jnp.float32),
        jax.ShapeDtypeStruct((8, 128), jnp.float32),
        jax.ShapeDtypeStruct((WROW, 128), jnp.int32),
        jax.ShapeDtypeStruct((WROW, 128), jnp.int32),
    ),
    in_specs=[
        pl.BlockSpec(memory_space=pltpu.SMEM),
        pl.BlockSpec(memory_space=pltpu.VMEM),
        pl.BlockSpec(memory_space=pltpu.VMEM),
    ],
    out_specs=(
        pl.BlockSpec(memory_space=pltpu.VMEM),
        pl.BlockSpec(memory_space=pltpu.VMEM),
        pl.BlockSpec(memory_space=pltpu.VMEM),
        pl.BlockSpec(memory_space=pltpu.VMEM),
    ),
)


def _edge_body(x_hbm, tabd_hbm, bvec_hbm, src_hbm, dst_hbm, out_hbm,
               sidx_all, didx_all,
               sbuf0, sbuf1, dbuf0, dbuf1, wbuf0, wbuf1,
               sidx_u0, sidx_u1, didx_g0, didx_g1, didx_s0, didx_s1,
               bvecv, accum,
               sgs0, sgd0, ssc0, sgs1, sgd1, ssc1):
    c = lax.axis_index("c")
    s = lax.axis_index("s")
    wid = c * NS + s

    pltpu.sync_copy(bvec_hbm, bvecv)
    bvec = bvecv[0, pl.ds(0, L)]
    nbvec = bvecv[1, pl.ds(0, L)]

    pltpu.sync_copy(src_hbm.at[pl.ds(wid * IROWS, IROWS), :], sidx_all)
    pltpu.sync_copy(dst_hbm.at[pl.ds(wid * IROWS, IROWS), :], didx_all)

    zero = jnp.zeros((L,), jnp.float32)

    def zrow(j, carry):
        for t in range(W // L):
            wbuf0[j, pl.ds(t * L, L)] = zero
        return carry

    lax.fori_loop(0, K, zrow, 0)
    for b in range(RPW // K):
        pltpu.sync_copy(wbuf0, accum.at[pl.ds(s * RPW + b * K, K), :])
    plsc.subcore_barrier()

    lomask = jnp.full((L,), 0xFFFF, jnp.int32)
    sh16 = jnp.full((L,), 16, jnp.int32)
    ones16 = jnp.ones((L,), jnp.float32)
    magic = jnp.full((L,), 0x5F3759DF, jnp.int32)
    sh1 = jnp.full((L,), 1, jnp.int32)
    half = jnp.full((L,), 0.5, jnp.float32)
    threehalf = jnp.full((L,), 1.5, jnp.float32)

    def expand(ci, packed_all, ubuf):
        v = packed_all[ci // 8, pl.ds((ci % 8) * L, L)]
        ubuf[0, pl.ds(0, L)] = v & lomask
        ubuf[0, pl.ds(L, L)] = lax.shift_right_logical(v, sh16)

    def gstart(ci, sidx_u, didx_g, sbuf, dbuf, sem_s, sem_d):
        expand(ci, sidx_all, sidx_u)
        expand(ci, didx_all, didx_g)
        pltpu.async_copy(x_hbm.at[sidx_u.at[0]], sbuf, sem_s)
        pltpu.async_copy(tabd_hbm.at[didx_g.at[0]], dbuf, sem_d)

    def gwait(sbuf, dbuf, sem_s, sem_d):
        pltpu.make_async_copy(x_hbm.at[pl.ds(0, K), :], sbuf, sem_s).wait()
        pltpu.make_async_copy(tabd_hbm.at[pl.ds(0, K), :], dbuf, sem_d).wait()

    def sstart(ci, wbuf, didx_s, sem):
        expand(ci, didx_all, didx_s)
        pltpu.async_copy(wbuf, accum.at[didx_s.at[0]], sem, add=True)

    def swait(wbuf, sem):
        pltpu.make_async_copy(out_hbm.at[0, pl.ds(0, K), :], wbuf, sem).wait()

    def compute(sbuf, dbuf, wbuf):
        @plsc.parallel_loop(0, K, unroll=4)
        def edge(e):
            sv = [sbuf[e, pl.ds(t * L, L)] for t in range(D // L)]
            dv = [dbuf[e, pl.ds(t * L, L)] for t in range(D // L)]
            acc0 = sv[0] * dv[0]
            acc1 = sv[1] * dv[1]
            acc2 = sv[2] * dv[2]
            acc3 = sv[3] * dv[3]
            ss0 = sv[0] * sv[0]
            ss1 = sv[1] * sv[1]
            ss2 = sv[2] * sv[2]
            ss3 = sv[3] * sv[3]
            for t in range(4, D // L, 4):
                acc0 = acc0 + sv[t] * dv[t]
                acc1 = acc1 + sv[t + 1] * dv[t + 1]
                acc2 = acc2 + sv[t + 2] * dv[t + 2]
                acc3 = acc3 + sv[t + 3] * dv[t + 3]
                ss0 = ss0 + sv[t] * sv[t]
                ss1 = ss1 + sv[t + 1] * sv[t + 1]
                ss2 = ss2 + sv[t + 2] * sv[t + 2]
                ss3 = ss3 + sv[t + 3] * sv[t + 3]
            dot = jnp.sum((acc0 + acc1) + (acc2 + acc3))
            ss = jnp.sum((ss0 + ss1) + (ss2 + ss3))
            ssv = jnp.maximum(ss * ones16, 1e-24)
            y = lax.bitcast_convert_type(
                magic - lax.shift_right_logical(
                    lax.bitcast_convert_type(ssv, jnp.int32), sh1),
                jnp.float32)
            hs = half * ssv
            y = y * (threehalf - hs * y * y)
            y = y * (threehalf - hs * y * y)
            y = y * (threehalf - hs * y * y)
            w = jnp.exp((dot * bvec) * y + nbvec)
            for t in range(D // L):
                wbuf[e, pl.ds(t * L, L)] = sv[t] * w
            wbuf[e, pl.ds(D, L)] = w

    gstart(0, sidx_u0, didx_g0, sbuf0, dbuf0, sgs0, sgd0)

    gwait(sbuf0, dbuf0, sgs0, sgd0)
    gstart(1, sidx_u1, didx_g1, sbuf1, dbuf1, sgs1, sgd1)
    compute(sbuf0, dbuf0, wbuf0)
    sstart(0, wbuf0, didx_s0, ssc0)

    gwait(sbuf1, dbuf1, sgs1, sgd1)
    gstart(2, sidx_u0, didx_g0, sbuf0, dbuf0, sgs0, sgd0)
    compute(sbuf1, dbuf1, wbuf1)
    sstart(1, wbuf1, didx_s1, ssc1)

    def pair(i, carry):
        ci = 2 * i
        gwait(sbuf0, dbuf0, sgs0, sgd0)
        swait(wbuf0, ssc0)
        gstart(ci + 1, sidx_u1, didx_g1, sbuf1, dbuf1, sgs1, sgd1)
        compute(sbuf0, dbuf0, wbuf0)
        sstart(ci, wbuf0, didx_s0, ssc0)

        gwait(sbuf1, dbuf1, sgs1, sgd1)
        swait(wbuf1, ssc1)
        gstart(ci + 2, sidx_u0, didx_g0, sbuf0, dbuf0, sgs0, sgd0)
        compute(sbuf1, dbuf1, wbuf1)
        sstart(ci + 1, wbuf1, didx_s1, ssc1)
        return carry

    lax.fori_loop(1, NCHUNK // 2 - 1, pair, 0)

    gwait(sbuf0, dbuf0, sgs0, sgd0)
    swait(wbuf0, ssc0)
    gstart(NCHUNK - 1, sidx_u1, didx_g1, sbuf1, dbuf1, sgs1, sgd1)
    compute(sbuf0, dbuf0, wbuf0)
    sstart(NCHUNK - 2, wbuf0, didx_s0, ssc0)

    gwait(sbuf1, dbuf1, sgs1, sgd1)
    swait(wbuf1, ssc1)
    compute(sbuf1, dbuf1, wbuf1)
    sstart(NCHUNK - 1, wbuf1, didx_s1, ssc1)

    swait(wbuf0, ssc0)
    swait(wbuf1, ssc1)
    plsc.subcore_barrier()
    pltpu.sync_copy(accum.at[pl.ds(s * RPW, RPW), :],
                    out_hbm.at[c, pl.ds(s * RPW, RPW), :])


_edge = pl.kernel(
    _edge_body,
    out_type=jax.ShapeDtypeStruct((NC, NPAD, W), jnp.float32),
    mesh=plsc.VectorSubcoreMesh(core_axis_name="c", subcore_axis_name="s"),
    compiler_params=pltpu.CompilerParams(
        needs_layout_passes=False, use_tc_tiling_on_sc=False),
    scratch_types=[
        pltpu.VMEM((IROWS, 128), jnp.int32),
        pltpu.VMEM((IROWS, 128), jnp.int32),
        pltpu.VMEM((K, D), jnp.float32),
        pltpu.VMEM((K, D), jnp.float32),
        pltpu.VMEM((K, D), jnp.float32),
        pltpu.VMEM((K, D), jnp.float32),
        pltpu.VMEM((K, W), jnp.float32),
        pltpu.VMEM((K, W), jnp.float32),
        pltpu.VMEM((1, K), jnp.int32),
        pltpu.VMEM((1, K), jnp.int32),
        pltpu.VMEM((1, K), jnp.int32),
        pltpu.VMEM((1, K), jnp.int32),
        pltpu.VMEM((1, K), jnp.int32),
        pltpu.VMEM((1, K), jnp.int32),
        pltpu.VMEM((8, 128), jnp.float32),
        pltpu.VMEM_SHARED((NPAD, W), jnp.float32),
        pltpu.SemaphoreType.DMA,
        pltpu.SemaphoreType.DMA,
        pltpu.SemaphoreType.DMA,
        pltpu.SemaphoreType.DMA,
        pltpu.SemaphoreType.DMA,
        pltpu.SemaphoreType.DMA,
    ],
)


def _combine_body(beta_ref, x_ref, p_ref, o_ref):
    x = x_ref[...]
    b = beta_ref[0]
    s2 = jnp.sum(x * x, axis=1, keepdims=True)
    nrm = jnp.maximum(jnp.sqrt(s2), 1e-12)
    xn2 = s2 / (nrm * nrm)
    selfw = jnp.exp(b * xn2 - jnp.abs(b))
    num = p_ref[0, 0:N, 0:D] + p_ref[1, 0:N, 0:D] + selfw * x
    den = p_ref[0, 0:N, D:D + 1] + p_ref[1, 0:N, D:D + 1] + selfw
    o_ref[...] = num / den


_combine = pl.pallas_call(
    _combine_body,
    out_shape=jax.ShapeDtypeStruct((N, D), jnp.float32),
    in_specs=[
        pl.BlockSpec(memory_space=pltpu.SMEM),
        pl.BlockSpec(memory_space=pltpu.VMEM),
        pl.BlockSpec(memory_space=pltpu.VMEM),
    ],
    out_specs=pl.BlockSpec(memory_space=pltpu.VMEM),
)


def kernel(x, edge_index, beta):
    tabd, bvec, srcp, dstp = _prep(beta, x, edge_index.reshape(2, WREAL * 2, 128))
    partials = _edge(x, tabd, bvec, srcp, dstp)
    return _combine(beta, x, partials)
